# baseline (device time: 147722 ns/iter reference)
import jax
import jax.numpy as jnp
from jax import lax
from jax.experimental import pallas as pl
from jax.experimental.pallas import tpu as pltpu

N_DEV = 16


def kernel(x, w_mat):
    m, _ = x.shape
    _, n = w_mat.shape
    chunk = m // N_DEV

    def body(x_ref, w_ref, out_ref, comm_ref, send_sems, recv_sems):
        me = lax.axis_index("i")
        left = (me - 1) % N_DEV
        right = (me + 1) % N_DEV

        barrier_sem = pltpu.get_barrier_semaphore()
        for nbr in (left, right):
            pl.semaphore_signal(
                barrier_sem, inc=1,
                device_id=(nbr,), device_id_type=pl.DeviceIdType.MESH,
            )
        pl.semaphore_wait(barrier_sem, 2)

        out_ref[...] = jnp.dot(
            x_ref[...], w_ref[...], preferred_element_type=jnp.float32
        )

        for s in range(N_DEV - 1):
            slot = s % N_DEV
            send_idx = (me - s) % N_DEV
            recv_idx = (me - s - 1) % N_DEV
            rdma = pltpu.make_async_remote_copy(
                src_ref=out_ref.at[pl.ds(send_idx * chunk, chunk)],
                dst_ref=comm_ref.at[slot],
                send_sem=send_sems.at[slot],
                recv_sem=recv_sems.at[slot],
                device_id=(right,),
                device_id_type=pl.DeviceIdType.MESH,
            )
            rdma.start()
            rdma.wait()
            out_ref[pl.ds(recv_idx * chunk, chunk)] = (
                out_ref[pl.ds(recv_idx * chunk, chunk)] + comm_ref[slot]
            )

        for g in range(N_DEV - 1):
            slot = (N_DEV - 1 + g) % N_DEV
            send_idx = (me + 1 - g) % N_DEV
            recv_idx = (me - g) % N_DEV
            rdma = pltpu.make_async_remote_copy(
                src_ref=out_ref.at[pl.ds(send_idx * chunk, chunk)],
                dst_ref=comm_ref.at[slot],
                send_sem=send_sems.at[slot],
                recv_sem=recv_sems.at[slot],
                device_id=(right,),
                device_id_type=pl.DeviceIdType.MESH,
            )
            rdma.start()
            rdma.wait()
            out_ref[pl.ds(recv_idx * chunk, chunk)] = comm_ref[slot]

    return pl.pallas_call(
        body,
        out_shape=jax.ShapeDtypeStruct((m, n), jnp.float32),
        in_specs=[
            pl.BlockSpec(memory_space=pltpu.VMEM),
            pl.BlockSpec(memory_space=pltpu.VMEM),
        ],
        out_specs=pl.BlockSpec(memory_space=pltpu.VMEM),
        scratch_shapes=[
            pltpu.VMEM((N_DEV, chunk, n), jnp.float32),
            pltpu.SemaphoreType.DMA((N_DEV,)),
            pltpu.SemaphoreType.DMA((N_DEV,)),
        ],
        compiler_params=pltpu.CompilerParams(collective_id=0),
    )(x, w_mat)


# device time: 101273 ns/iter; 1.4587x vs baseline; 1.4587x over previous
import jax
import jax.numpy as jnp
from jax import lax
from jax.experimental import pallas as pl
from jax.experimental.pallas import tpu as pltpu

N_DEV = 16
PIECES = 2


def kernel(x, w_mat):
    m, _ = x.shape
    _, n = w_mat.shape
    chunk = m // N_DEV
    half = n // 2
    width = half // PIECES
    n_streams = 2 * PIECES
    n_hops = 2 * (N_DEV - 1)

    streams = [(+1, p * width) for p in range(PIECES)] + [
        (-1, half + p * width) for p in range(PIECES)
    ]

    def body(x_ref, w_ref, out_ref, comm_ref, send_sems, recv_sems):
        me = lax.axis_index("i")
        left = (me - 1) % N_DEV
        right = (me + 1) % N_DEV

        barrier_sem = pltpu.get_barrier_semaphore()
        for nbr in (left, right):
            pl.semaphore_signal(
                barrier_sem, inc=1,
                device_id=(nbr,), device_id_type=pl.DeviceIdType.MESH,
            )
        pl.semaphore_wait(barrier_sem, 2)

        out_ref[...] = jnp.dot(
            x_ref[...], w_ref[...], preferred_element_type=jnp.float32
        )

        def make_rdma(st, sigma, col0, h):
            slot = h % N_DEV
            if h < N_DEV - 1:
                send_idx = (me - sigma * h) % N_DEV
            else:
                g = h - (N_DEV - 1)
                send_idx = (me + sigma * (1 - g)) % N_DEV
            return pltpu.make_async_remote_copy(
                src_ref=out_ref.at[
                    pl.ds(send_idx * chunk, chunk), pl.ds(col0, width)
                ],
                dst_ref=comm_ref.at[st, slot],
                send_sem=send_sems.at[st, slot],
                recv_sem=recv_sems.at[st, slot],
                device_id=((me + sigma) % N_DEV,),
                device_id_type=pl.DeviceIdType.MESH,
            )

        def process(st, sigma, col0, h):
            slot = h % N_DEV
            if h < N_DEV - 1:
                recv_idx = (me - sigma * (h + 1)) % N_DEV
            else:
                g = h - (N_DEV - 1)
                recv_idx = (me - sigma * g) % N_DEV
            rows = pl.ds(recv_idx * chunk, chunk)
            cols = pl.ds(col0, width)
            if h < N_DEV - 1:
                out_ref[rows, cols] = out_ref[rows, cols] + comm_ref[st, slot]
            else:
                out_ref[rows, cols] = comm_ref[st, slot]

        for h in range(n_hops):
            for st, (sigma, col0) in enumerate(streams):
                if h > 0:
                    prev = make_rdma(st, sigma, col0, h - 1)
                    prev.wait_send()
                    prev.wait_recv()
                    process(st, sigma, col0, h - 1)
                make_rdma(st, sigma, col0, h).start()

        for st, (sigma, col0) in enumerate(streams):
            last = make_rdma(st, sigma, col0, n_hops - 1)
            last.wait_send()
            last.wait_recv()
            process(st, sigma, col0, n_hops - 1)

    return pl.pallas_call(
        body,
        out_shape=jax.ShapeDtypeStruct((m, n), jnp.float32),
        in_specs=[
            pl.BlockSpec(memory_space=pltpu.VMEM),
            pl.BlockSpec(memory_space=pltpu.VMEM),
        ],
        out_specs=pl.BlockSpec(memory_space=pltpu.VMEM),
        scratch_shapes=[
            pltpu.VMEM((n_streams, N_DEV, chunk, width), jnp.float32),
            pltpu.SemaphoreType.DMA((n_streams, N_DEV)),
            pltpu.SemaphoreType.DMA((n_streams, N_DEV)),
        ],
        compiler_params=pltpu.CompilerParams(collective_id=0),
    )(x, w_mat)


# device time: 82321 ns/iter; 1.7945x vs baseline; 1.2302x over previous
import jax
import jax.numpy as jnp
from jax import lax
from jax.experimental import pallas as pl
from jax.experimental.pallas import tpu as pltpu

N_DEV = 16
PIECES = 4

RING = [0, 1, 5, 9, 13, 14, 10, 6, 2, 3, 7, 11, 15, 12, 8, 4]
RANK_OF_MESH = [RING.index(m) for m in range(N_DEV)]
NEXT_BY_MESH = [RING[(RANK_OF_MESH[m] + 1) % N_DEV] for m in range(N_DEV)]
PREV_BY_MESH = [RING[(RANK_OF_MESH[m] - 1) % N_DEV] for m in range(N_DEV)]

N_STEPS = 17


def _messages():
    cw = []
    for s in range(7):
        cw.append((s, s, s + 1, 7 - s, 6 - s, True))
    for u in range(8):
        cw.append((7 + u, 8 + u, 9 + u, -u, -1 - u, False))
    ccw = []
    for t in range(8):
        ccw.append((t, t, t + 1, t - 8, t - 7, True))
    for v in range(7):
        ccw.append((8 + v, 8 + v, 9 + v, v, v + 1, False))
    return {+1: cw, -1: ccw}


def _lut(table, idx):
    val = jnp.int32(table[0])
    for j in range(1, len(table)):
        val = jnp.where(idx == j, jnp.int32(table[j]), val)
    return val


def kernel(x, w_mat):
    m, _ = x.shape
    _, n = w_mat.shape
    chunk = m // N_DEV
    width = n // PIECES
    streams = []
    for p in range(PIECES):
        streams.append((+1, p))
        streams.append((-1, p))
    msgs = _messages()

    def body(x_ref, w_ref, out_ref, comm_ref, send_sems, recv_sems):
        me = lax.axis_index("i")
        r = _lut(RANK_OF_MESH, me)
        nxt = _lut(NEXT_BY_MESH, me)
        prv = _lut(PREV_BY_MESH, me)

        out_ref[...] = jnp.dot(
            x_ref[...], w_ref[...], preferred_element_type=jnp.float32
        )

        barrier_sem = pltpu.get_barrier_semaphore()
        for nbr in (nxt, prv):
            pl.semaphore_signal(
                barrier_sem, inc=1,
                device_id=(nbr,), device_id_type=pl.DeviceIdType.MESH,
            )
        pl.semaphore_wait(barrier_sem, 2)

        def make_rdma(st, sigma, p, msg):
            slot, _, _, send_off, _, _ = msg
            send_chunk = (r + sigma * 0 + send_off) % N_DEV
            return pltpu.make_async_remote_copy(
                src_ref=out_ref.at[
                    pl.ds(send_chunk * chunk, chunk), pl.ds(p * width, width)
                ],
                dst_ref=comm_ref.at[st, slot],
                send_sem=send_sems.at[st, slot],
                recv_sem=recv_sems.at[st, slot],
                device_id=(nxt if sigma > 0 else prv,),
                device_id_type=pl.DeviceIdType.MESH,
            )

        for k in range(N_STEPS):
            for st, (sigma, p) in enumerate(streams):
                for msg in msgs[sigma]:
                    slot, _, proc_k, _, recv_off, add = msg
                    if proc_k != k:
                        continue
                    d = make_rdma(st, sigma, p, msg)
                    d.wait_send()
                    d.wait_recv()
                    recv_chunk = (r + recv_off) % N_DEV
                    rows = pl.ds(recv_chunk * chunk, chunk)
                    cols = pl.ds(p * width, width)
                    if add:
                        out_ref[rows, cols] = (
                            out_ref[rows, cols] + comm_ref[st, slot]
                        )
                    else:
                        out_ref[rows, cols] = comm_ref[st, slot]
            for st, (sigma, p) in enumerate(streams):
                for msg in msgs[sigma]:
                    if msg[1] == k:
                        make_rdma(st, sigma, p, msg).start()

    n_streams = len(streams)
    return pl.pallas_call(
        body,
        out_shape=jax.ShapeDtypeStruct((m, n), jnp.float32),
        in_specs=[
            pl.BlockSpec(memory_space=pltpu.VMEM),
            pl.BlockSpec(memory_space=pltpu.VMEM),
        ],
        out_specs=pl.BlockSpec(memory_space=pltpu.VMEM),
        scratch_shapes=[
            pltpu.VMEM((n_streams, 15, chunk, width), jnp.float32),
            pltpu.SemaphoreType.DMA((n_streams, 15)),
            pltpu.SemaphoreType.DMA((n_streams, 15)),
        ],
        compiler_params=pltpu.CompilerParams(collective_id=0),
    )(x, w_mat)


# device time: 55046 ns/iter; 2.6836x vs baseline; 1.4955x over previous
import jax
import jax.numpy as jnp
from jax import lax
from jax.experimental import pallas as pl
from jax.experimental.pallas import tpu as pltpu

N_DEV = 16
PIECES = 4

RING = [0, 1, 5, 9, 13, 14, 10, 6, 2, 3, 7, 11, 15, 12, 8, 4]
RANK_OF_MESH = [RING.index(m) for m in range(N_DEV)]
NEXT_BY_MESH = [RING[(RANK_OF_MESH[m] + 1) % N_DEV] for m in range(N_DEV)]
PREV_BY_MESH = [RING[(RANK_OF_MESH[m] - 1) % N_DEV] for m in range(N_DEV)]

N_STEPS = 17


def _messages():
    cw = []
    for s in range(7):
        cw.append((s, s, s + 1, 7 - s, 6 - s, True))
    for u in range(8):
        cw.append((7 + u, 8 + u, 9 + u, -u, -1 - u, False))
    ccw = []
    for t in range(8):
        ccw.append((t, t, t + 1, t - 8, t - 7, True))
    for v in range(7):
        ccw.append((8 + v, 8 + v, 9 + v, v, v + 1, False))
    return {+1: cw, -1: ccw}


def _lut(table, idx):
    val = jnp.int32(table[0])
    for j in range(1, len(table)):
        val = jnp.where(idx == j, jnp.int32(table[j]), val)
    return val


def kernel(x, w_mat):
    m, _ = x.shape
    _, n = w_mat.shape
    chunk = m // N_DEV
    width = n // PIECES
    streams = []
    for p in range(PIECES):
        streams.append((-1, p))
        streams.append((+1, p))
    msgs = _messages()

    def body(x_ref, w_ref, out_ref, comm_ref, send_sems, recv_sems):
        me = lax.axis_index("i")
        r = _lut(RANK_OF_MESH, me)
        nxt = _lut(NEXT_BY_MESH, me)
        prv = _lut(PREV_BY_MESH, me)

        out_ref[...] = jnp.dot(
            x_ref[...], w_ref[...], preferred_element_type=jnp.float32
        )

        barrier_sem = pltpu.get_barrier_semaphore()
        for nbr in (nxt, prv):
            pl.semaphore_signal(
                barrier_sem, inc=1,
                device_id=(nbr,), device_id_type=pl.DeviceIdType.MESH,
            )
        pl.semaphore_wait(barrier_sem, 2)

        def make_rdma(st, sigma, p, msg, for_recv):
            slot, _, _, send_off, recv_off, reduce = msg
            cols = pl.ds(p * width, width)
            src_chunk = (r + send_off) % N_DEV
            src = out_ref.at[pl.ds(src_chunk * chunk, chunk), cols]
            if reduce:
                dst = comm_ref.at[st, slot]
            elif for_recv:
                dst_chunk = (r + recv_off) % N_DEV
                dst = out_ref.at[pl.ds(dst_chunk * chunk, chunk), cols]
            else:
                dst = out_ref.at[pl.ds(src_chunk * chunk, chunk), cols]
            return pltpu.make_async_remote_copy(
                src_ref=src,
                dst_ref=dst,
                send_sem=send_sems.at[st, slot],
                recv_sem=recv_sems.at[st, slot],
                device_id=(nxt if sigma > 0 else prv,),
                device_id_type=pl.DeviceIdType.MESH,
            )

        for k in range(N_STEPS):
            for st, (sigma, p) in enumerate(streams):
                for msg in msgs[sigma]:
                    slot, _, proc_k, _, recv_off, reduce = msg
                    if proc_k != k:
                        continue
                    d = make_rdma(st, sigma, p, msg, for_recv=True)
                    d.wait_send()
                    d.wait_recv()
                    if reduce:
                        recv_chunk = (r + recv_off) % N_DEV
                        rows = pl.ds(recv_chunk * chunk, chunk)
                        cols = pl.ds(p * width, width)
                        out_ref[rows, cols] = (
                            out_ref[rows, cols] + comm_ref[st, slot]
                        )
                for msg in msgs[sigma]:
                    if msg[1] == k:
                        make_rdma(st, sigma, p, msg, for_recv=False).start()

    n_streams = len(streams)
    return pl.pallas_call(
        body,
        out_shape=jax.ShapeDtypeStruct((m, n), jnp.float32),
        in_specs=[
            pl.BlockSpec(memory_space=pltpu.VMEM),
            pl.BlockSpec(memory_space=pltpu.VMEM),
        ],
        out_specs=pl.BlockSpec(memory_space=pltpu.VMEM),
        scratch_shapes=[
            pltpu.VMEM((n_streams, 8, chunk, width), jnp.float32),
            pltpu.SemaphoreType.DMA((n_streams, 15)),
            pltpu.SemaphoreType.DMA((n_streams, 15)),
        ],
        compiler_params=pltpu.CompilerParams(collective_id=0),
    )(x, w_mat)
